# baseline (speedup 1.0000x reference)
import functools
import numpy as _np

import jax
import jax.numpy as jnp
from jax import lax
from jax.experimental import pallas as pl
from jax.experimental.pallas import tpu as pltpu
from jax.experimental.pallas import tpu_sc as plsc

_EPS = 1e-5
_NC = 2
_NS = 16
_NW = _NC * _NS


def _sc_degrees(dst_r, ones128, z64, n, npad, rpt):
    nw, k, _ = dst_r.shape

    @functools.partial(
        pl.kernel,
        out_type=jax.ShapeDtypeStruct((_NC, npad, 128), jnp.float32),
        mesh=plsc.VectorSubcoreMesh(core_axis_name="c", subcore_axis_name="s"),
        scratch_types=[
            pltpu.VMEM((k, 128), jnp.int32),
            pltpu.VMEM((128, 128), jnp.float32),
            pltpu.VMEM((64, 128), jnp.float32),
            pltpu.VMEM_SHARED((npad, 128), jnp.float32),
            pltpu.SemaphoreType.DMA,
        ],
    )
    def deg_kernel(dst_hbm, ones_hbm, z_hbm, degp, idx_v, ones_v, zbuf,
                   deg_s, sem):
        cid = lax.axis_index("c")
        sid = lax.axis_index("s")
        wid = cid * _NS + sid
        zrows = npad // _NS
        pltpu.sync_copy(z_hbm, zbuf)
        for t in range(zrows // 64):
            pltpu.sync_copy(zbuf, deg_s.at[pl.ds(sid * zrows + t * 64, 64)])
        pltpu.sync_copy(ones_hbm, ones_v)
        pltpu.sync_copy(dst_hbm.at[wid], idx_v)
        plsc.subcore_barrier()

        def body(j, _):
            pltpu.sync_copy(ones_v, deg_s.at[idx_v.at[j]], add=True)
            return ()

        lax.fori_loop(0, k, body, ())
        plsc.subcore_barrier()

        for t in range(zrows // 64):
            base = sid * zrows + t * 64
            pltpu.sync_copy(deg_s.at[pl.ds(base, 64)], zbuf)
            pltpu.sync_copy(zbuf, degp.at[cid, pl.ds(base, 64)])

    return deg_kernel(dst_r, ones128, z64)


def _sc_scatter(g, src_r, dst_r, z128, n, npad, rpt):
    nw, k, _ = src_r.shape
    ph = (k + 1) // 2

    @functools.partial(
        pl.kernel,
        out_type=jax.ShapeDtypeStruct((_NC, npad, 128), jnp.float32),
        mesh=plsc.VectorSubcoreMesh(core_axis_name="c", subcore_axis_name="s"),
        scratch_types=[
            pltpu.VMEM((ph, 128), jnp.int32),
            pltpu.VMEM((ph, 128), jnp.int32),
            pltpu.VMEM((128, 128), jnp.float32),
            pltpu.VMEM((128, 128), jnp.float32),
            pltpu.VMEM_SHARED((npad, 128), jnp.float32),
            pltpu.SemaphoreType.DMA,
            pltpu.SemaphoreType.DMA,
        ],
    )
    def scat_kernel(g_hbm, src_hbm, dst_hbm, z_hbm, accp, sidx, didx, r0, r1,
                    acc_s, sem0, sem1):
        cid = lax.axis_index("c")
        sid = lax.axis_index("s")
        wid = cid * _NS + sid
        zrows = npad // _NS
        pltpu.sync_copy(z_hbm, r0)
        for t in range(zrows // 128):
            pltpu.sync_copy(r0, acc_s.at[pl.ds(sid * zrows + t * 128, 128)])
        plsc.subcore_barrier()

        for phase in range(2):
            base = phase * ph
            ke = min(k - base, ph)
            pltpu.sync_copy(src_hbm.at[wid, pl.ds(base, ke)],
                            sidx.at[pl.ds(0, ke)])
            pltpu.sync_copy(dst_hbm.at[wid, pl.ds(base, ke)],
                            didx.at[pl.ds(0, ke)])
            pltpu.async_copy(g_hbm.at[sidx.at[0]], r0, sem0)

            def body(j, _):
                @pl.when((j % 2 == 0) & (j + 1 < ke))
                def _f0():
                    pltpu.async_copy(g_hbm.at[sidx.at[j + 1]], r1, sem1)

                @pl.when((j % 2 == 1) & (j + 1 < ke))
                def _f1():
                    pltpu.async_copy(g_hbm.at[sidx.at[j + 1]], r0, sem0)

                @pl.when(j % 2 == 0)
                def _s0():
                    pltpu.make_async_copy(g_hbm.at[sidx.at[j]], r0, sem0).wait()
                    pltpu.sync_copy(r0, acc_s.at[didx.at[j]], add=True)

                @pl.when(j % 2 == 1)
                def _s1():
                    pltpu.make_async_copy(g_hbm.at[sidx.at[j]], r1, sem1).wait()
                    pltpu.sync_copy(r1, acc_s.at[didx.at[j]], add=True)

                return ()

            lax.fori_loop(0, ke, body, ())
        plsc.subcore_barrier()

        for t in range(zrows // 128):
            b = sid * zrows + t * 128
            pltpu.sync_copy(acc_s.at[pl.ds(b, 128)], r0)
            pltpu.sync_copy(r0, accp.at[cid, pl.ds(b, 128)])

    return scat_kernel(g, src_r, dst_r, z128)


def _tc_pre(xp, degp2, in_w, in_b, bng, bnb, w0, n, r):
    nblk = n // r
    bns = 1.0 / float(_np.sqrt(1.0 + _EPS))

    def body(x_ref, d_ref, w_ref, b_ref, g_ref, bb_ref, w0_ref,
             g0_ref, dinv_ref):
        deg = jnp.sum(d_ref[...], axis=1, keepdims=True) + 1.0
        dinv = lax.rsqrt(deg)
        h = jnp.dot(x_ref[...], w_ref[...], preferred_element_type=jnp.float32)
        h = (h + b_ref[...]) * (g_ref[...] * bns) + bb_ref[...]
        h = jnp.maximum(h, 0.0)
        g0_ref[...] = jnp.dot(dinv * h, w0_ref[...],
                              preferred_element_type=jnp.float32)
        dinv_ref[...] = dinv

    return pl.pallas_call(
        body,
        grid=(nblk,),
        in_specs=[
            pl.BlockSpec((r, xp.shape[1]), lambda i: (i, 0)),
            pl.BlockSpec((r, 2), lambda i: (i, 0)),
            pl.BlockSpec(in_w.shape, lambda i: (0, 0)),
            pl.BlockSpec((1, 128), lambda i: (0, 0)),
            pl.BlockSpec((1, 128), lambda i: (0, 0)),
            pl.BlockSpec((1, 128), lambda i: (0, 0)),
            pl.BlockSpec((128, 128), lambda i: (0, 0)),
        ],
        out_specs=[
            pl.BlockSpec((r, 128), lambda i: (i, 0)),
            pl.BlockSpec((r, 1), lambda i: (i, 0)),
        ],
        out_shape=[
            jax.ShapeDtypeStruct((n, 128), jnp.float32),
            jax.ShapeDtypeStruct((n, 1), jnp.float32),
        ],
    )(xp, degp2, in_w, in_b, bng, bnb, w0)


def _tc_mid(accp, g, hprev, dinv, b_gcn, bng, bnb, w_next, n, r):
    nblk = n // r
    bns = 1.0 / float(_np.sqrt(1.0 + _EPS))
    residual = hprev is not None

    def body(*refs):
        if residual:
            (acc_ref, g_ref, h_ref, dinv_ref, b_ref, bg_ref, bb_ref, w_ref,
             h_out, g_out) = refs
        else:
            (acc_ref, g_ref, dinv_ref, b_ref, bg_ref, bb_ref, w_ref,
             h_out, g_out) = refs
        m = acc_ref[0] + acc_ref[1]
        dinv = dinv_ref[...]
        conv = dinv * (m + g_ref[...]) + b_ref[...]
        hn = jnp.maximum(conv * (bg_ref[...] * bns) + bb_ref[...], 0.0)
        if residual:
            hn = hn + h_ref[...]
        h_out[...] = hn
        g_out[...] = jnp.dot(dinv * hn, w_ref[...],
                             preferred_element_type=jnp.float32)

    in_specs = [pl.BlockSpec((2, r, 128), lambda i: (0, i, 0)),
                pl.BlockSpec((r, 128), lambda i: (i, 0))]
    args = [accp, g]
    if residual:
        in_specs.append(pl.BlockSpec((r, 128), lambda i: (i, 0)))
        args.append(hprev)
    in_specs += [
        pl.BlockSpec((r, 1), lambda i: (i, 0)),
        pl.BlockSpec((1, 128), lambda i: (0, 0)),
        pl.BlockSpec((1, 128), lambda i: (0, 0)),
        pl.BlockSpec((1, 128), lambda i: (0, 0)),
        pl.BlockSpec((128, 128), lambda i: (0, 0)),
    ]
    args += [dinv, b_gcn, bng, bnb, w_next]
    return pl.pallas_call(
        body,
        grid=(nblk,),
        in_specs=in_specs,
        out_specs=[
            pl.BlockSpec((r, 128), lambda i: (i, 0)),
            pl.BlockSpec((r, 128), lambda i: (i, 0)),
        ],
        out_shape=[
            jax.ShapeDtypeStruct((n, 128), jnp.float32),
            jax.ShapeDtypeStruct((n, 128), jnp.float32),
        ],
    )(*args)


def _tc_final(accp, g, hprev, dinv, batch_r, b_gcn, bng, bnb, cls, n, r,
              num_graphs):
    nblk = n // r
    bns = 1.0 / float(_np.sqrt(1.0 + _EPS))
    (c0w, c0b, c0g, c0bb, c1w, c1b, c1g, c1bb, c2w, c2b) = cls

    def body(acc_ref, g_ref, h_ref, dinv_ref, b_ref, bg_ref, bb_ref,
             batch_ref, c0w_r, c0b_r, c0g_r, c0bb_r, c1w_r, c1b_r, c1g_r,
             c1bb_r, c2w_r, c2b_r, out_ref, sums, counts):
        i = pl.program_id(0)

        @pl.when(i == 0)
        def _init():
            sums[...] = jnp.zeros_like(sums)
            counts[...] = jnp.zeros_like(counts)

        m = acc_ref[0] + acc_ref[1]
        dinv = dinv_ref[...]
        conv = dinv * (m + g_ref[...]) + b_ref[...]
        hn = jnp.maximum(conv * (bg_ref[...] * bns) + bb_ref[...], 0.0)
        h3 = hn + h_ref[...]

        seg = batch_ref[0]
        gid = lax.broadcasted_iota(jnp.int32, (num_graphs, r), 0)
        onehot = jnp.where(gid == seg, 1.0, 0.0)
        sums[...] += jnp.dot(onehot, h3, preferred_element_type=jnp.float32)
        cnt = jnp.sum(onehot, axis=1, keepdims=True)
        counts[...] += jnp.broadcast_to(cnt, counts.shape)

        @pl.when(i == nblk - 1)
        def _fin():
            gemb = sums[...] / jnp.maximum(counts[...], 1.0)
            z = jnp.dot(gemb, c0w_r[...], preferred_element_type=jnp.float32)
            z = (z + c0b_r[...]) * (c0g_r[...] * bns) + c0bb_r[...]
            z = jnp.maximum(z, 0.0)
            z = jnp.dot(z, c1w_r[...], preferred_element_type=jnp.float32)
            z = (z + c1b_r[...]) * (c1g_r[...] * bns) + c1bb_r[...]
            z = jnp.maximum(z, 0.0)
            z = jnp.dot(z, c2w_r[...], preferred_element_type=jnp.float32)
            out_ref[...] = z + c2b_r[...]

    full = lambda a: pl.BlockSpec(a.shape, lambda i: tuple(0 for _ in a.shape))
    return pl.pallas_call(
        body,
        grid=(nblk,),
        in_specs=[
            pl.BlockSpec((2, r, 128), lambda i: (0, i, 0)),
            pl.BlockSpec((r, 128), lambda i: (i, 0)),
            pl.BlockSpec((r, 128), lambda i: (i, 0)),
            pl.BlockSpec((r, 1), lambda i: (i, 0)),
            pl.BlockSpec((1, 128), lambda i: (0, 0)),
            pl.BlockSpec((1, 128), lambda i: (0, 0)),
            pl.BlockSpec((1, 128), lambda i: (0, 0)),
            pl.BlockSpec((1, 1, r), lambda i: (i, 0, 0)),
            full(c0w), full(c0b), full(c0g), full(c0bb),
            full(c1w), full(c1b), full(c1g), full(c1bb),
            full(c2w), full(c2b),
        ],
        out_specs=pl.BlockSpec((num_graphs, 2), lambda i: (0, 0)),
        out_shape=jax.ShapeDtypeStruct((num_graphs, 2), jnp.float32),
        scratch_shapes=[
            pltpu.VMEM((num_graphs, 128), jnp.float32),
            pltpu.VMEM((num_graphs, 128), jnp.float32),
        ],
    )(accp, g, hprev, dinv, b_gcn, bng, bnb, batch_r,
      c0w, c0b, c0g, c0bb, c1w, c1b, c1g, c1bb, c2w, c2b)


def kernel(x, edge_index, batch, params):
    n, in_dim = x.shape
    e = edge_index.shape[1]
    num_graphs = 32
    r = 1000
    ept = e // _NW
    k = (ept + 127) // 128
    pad = k * 128 - ept
    rpt = n // _NS
    npad = ((n + 639) // 640) * 640

    src = edge_index[0].astype(jnp.int32)
    dst = edge_index[1].astype(jnp.int32)
    src_r = jnp.pad(src.reshape(_NW, ept), ((0, 0), (0, pad))).reshape(
        _NW, k, 128)
    dst_r = jnp.pad(dst.reshape(_NW, ept), ((0, 0), (0, pad)),
                    constant_values=n).reshape(_NW, k, 128)
    ones128 = jnp.ones((128, 128), jnp.float32)
    z64 = jnp.zeros((64, 128), jnp.float32)
    z128 = jnp.zeros((128, 128), jnp.float32)

    p = params
    row = lambda v: v.reshape(1, -1)

    degp = _sc_degrees(dst_r, ones128, z64, n, npad, rpt)[:, :n, :1]
    degp2 = jnp.transpose(degp, (1, 0, 2)).reshape(n, _NC)

    xp = jnp.pad(x, ((0, 0), (0, (-in_dim) % 8)))
    in_w = jnp.pad(p['in_W'], ((0, (-in_dim) % 8), (0, 0)))
    g0, dinv = _tc_pre(xp, degp2, in_w, row(p['in_b']), row(p['bn_in_g']),
                       row(p['bn_in_b']), p['gcn0_W'], n, r)

    acc = _sc_scatter(g0, src_r, dst_r, z128, n, npad, rpt)[:, :n]
    h1, g1 = _tc_mid(acc, g0, None, dinv, row(p['gcn0_b']), row(p['bn0_g']),
                     row(p['bn0_b']), p['gcn1_W'], n, r)

    acc = _sc_scatter(g1, src_r, dst_r, z128, n, npad, rpt)[:, :n]
    h2, g2 = _tc_mid(acc, g1, h1, dinv, row(p['gcn1_b']), row(p['bn1_g']),
                     row(p['bn1_b']), p['gcn2_W'], n, r)

    acc = _sc_scatter(g2, src_r, dst_r, z128, n, npad, rpt)[:, :n]
    batch_r = batch.astype(jnp.int32).reshape(n // r, 1, r)
    cls = (p['cl0_W'], row(p['cl0_b']), row(p['cbn0_g']), row(p['cbn0_b']),
           p['cl1_W'], row(p['cl1_b']), row(p['cbn1_g']), row(p['cbn1_b']),
           p['cl2_W'], row(p['cl2_b']))
    return _tc_final(acc, g2, h2, dinv, batch_r, row(p['gcn2_b']),
                     row(p['bn2_g']), row(p['bn2_b']), cls, n, r, num_graphs)


# speedup vs baseline: 1.0285x; 1.0285x over previous
import functools
import numpy as _np

import jax
import jax.numpy as jnp
from jax import lax
from jax.experimental import pallas as pl
from jax.experimental.pallas import tpu as pltpu
from jax.experimental.pallas import tpu_sc as plsc

_EPS = 1e-5
_NC = 2
_NS = 16
_NW = _NC * _NS


def _sc_degrees(dst_r, ones128, z64, n, npad, rpt):
    nw, k, _ = dst_r.shape

    @functools.partial(
        pl.kernel,
        out_type=jax.ShapeDtypeStruct((_NC, npad, 128), jnp.float32),
        mesh=plsc.VectorSubcoreMesh(core_axis_name="c", subcore_axis_name="s"),
        scratch_types=[
            pltpu.VMEM((k, 128), jnp.int32),
            pltpu.VMEM((128, 128), jnp.float32),
            pltpu.VMEM((64, 128), jnp.float32),
            pltpu.VMEM_SHARED((npad, 128), jnp.float32),
            pltpu.SemaphoreType.DMA,
        ],
    )
    def deg_kernel(dst_hbm, ones_hbm, z_hbm, degp, idx_v, ones_v, zbuf,
                   deg_s, sem):
        cid = lax.axis_index("c")
        sid = lax.axis_index("s")
        wid = cid * _NS + sid
        zrows = npad // _NS
        pltpu.sync_copy(z_hbm, zbuf)
        for t in range(zrows // 64):
            pltpu.sync_copy(zbuf, deg_s.at[pl.ds(sid * zrows + t * 64, 64)])
        pltpu.sync_copy(ones_hbm, ones_v)
        pltpu.sync_copy(dst_hbm.at[wid], idx_v)
        plsc.subcore_barrier()

        def body(j, _):
            pltpu.sync_copy(ones_v, deg_s.at[idx_v.at[j]], add=True)
            return ()

        lax.fori_loop(0, k, body, ())
        plsc.subcore_barrier()

        for t in range(zrows // 64):
            base = sid * zrows + t * 64
            pltpu.sync_copy(deg_s.at[pl.ds(base, 64)], zbuf)
            pltpu.sync_copy(zbuf, degp.at[cid, pl.ds(base, 64)])

    return deg_kernel(dst_r, ones128, z64)


def _sc_scatter(g, src_r, dst_r, z128, n, npad, rpt):
    nw, k, _ = src_r.shape
    ph = (k + 1) // 2

    @functools.partial(
        pl.kernel,
        out_type=jax.ShapeDtypeStruct((_NC, npad, 128), jnp.float32),
        mesh=plsc.VectorSubcoreMesh(core_axis_name="c", subcore_axis_name="s"),
        scratch_types=[
            pltpu.VMEM((ph, 128), jnp.int32),
            pltpu.VMEM((ph, 128), jnp.int32),
            pltpu.VMEM((128, 128), jnp.float32),
            pltpu.VMEM((128, 128), jnp.float32),
            pltpu.VMEM_SHARED((npad, 128), jnp.float32),
            pltpu.SemaphoreType.DMA,
            pltpu.SemaphoreType.DMA,
        ],
    )
    def scat_kernel(g_hbm, src_hbm, dst_hbm, z_hbm, accp, sidx, didx, r0, r1,
                    acc_s, sem0, sem1):
        cid = lax.axis_index("c")
        sid = lax.axis_index("s")
        wid = cid * _NS + sid
        zrows = npad // _NS
        pltpu.sync_copy(z_hbm, r0)
        for t in range(zrows // 128):
            pltpu.sync_copy(r0, acc_s.at[pl.ds(sid * zrows + t * 128, 128)])
        plsc.subcore_barrier()

        for phase in range(2):
            base = phase * ph
            ke = min(k - base, ph)
            pltpu.sync_copy(src_hbm.at[wid, pl.ds(base, ke)],
                            sidx.at[pl.ds(0, ke)])
            pltpu.sync_copy(dst_hbm.at[wid, pl.ds(base, ke)],
                            didx.at[pl.ds(0, ke)])
            pltpu.async_copy(g_hbm.at[sidx.at[0]], r0, sem0)

            def body(j, _):
                @pl.when((j % 2 == 0) & (j + 1 < ke))
                def _f0():
                    pltpu.async_copy(g_hbm.at[sidx.at[j + 1]], r1, sem1)

                @pl.when((j % 2 == 1) & (j + 1 < ke))
                def _f1():
                    pltpu.async_copy(g_hbm.at[sidx.at[j + 1]], r0, sem0)

                @pl.when(j % 2 == 0)
                def _s0():
                    pltpu.make_async_copy(g_hbm.at[sidx.at[j]], r0, sem0).wait()
                    pltpu.sync_copy(r0, acc_s.at[didx.at[j]], add=True)

                @pl.when(j % 2 == 1)
                def _s1():
                    pltpu.make_async_copy(g_hbm.at[sidx.at[j]], r1, sem1).wait()
                    pltpu.sync_copy(r1, acc_s.at[didx.at[j]], add=True)

                return ()

            lax.fori_loop(0, ke, body, ())
        plsc.subcore_barrier()

        for t in range(zrows // 128):
            b = sid * zrows + t * 128
            pltpu.sync_copy(acc_s.at[pl.ds(b, 128)], r0)
            pltpu.sync_copy(r0, accp.at[cid, pl.ds(b, 128)])

    return scat_kernel(g, src_r, dst_r, z128)


def _tc_pre(xp, degp2, in_w, in_b, bng, bnb, w0, n, r):
    nblk = n // r
    bns = 1.0 / float(_np.sqrt(1.0 + _EPS))

    def body(x_ref, d_ref, w_ref, b_ref, g_ref, bb_ref, w0_ref,
             g0_ref, dinv_ref):
        deg = jnp.sum(d_ref[...], axis=1, keepdims=True) + 1.0
        dinv = lax.rsqrt(deg)
        h = jnp.dot(x_ref[...], w_ref[...], preferred_element_type=jnp.float32)
        h = (h + b_ref[...]) * (g_ref[...] * bns) + bb_ref[...]
        h = jnp.maximum(h, 0.0)
        g0_ref[...] = jnp.dot(dinv * h, w0_ref[...],
                              preferred_element_type=jnp.float32)
        dinv_ref[...] = dinv

    return pl.pallas_call(
        body,
        grid=(nblk,),
        in_specs=[
            pl.BlockSpec((r, xp.shape[1]), lambda i: (i, 0)),
            pl.BlockSpec((r, 2), lambda i: (i, 0)),
            pl.BlockSpec(in_w.shape, lambda i: (0, 0)),
            pl.BlockSpec((1, 128), lambda i: (0, 0)),
            pl.BlockSpec((1, 128), lambda i: (0, 0)),
            pl.BlockSpec((1, 128), lambda i: (0, 0)),
            pl.BlockSpec((128, 128), lambda i: (0, 0)),
        ],
        out_specs=[
            pl.BlockSpec((r, 128), lambda i: (i, 0)),
            pl.BlockSpec((r, 1), lambda i: (i, 0)),
        ],
        out_shape=[
            jax.ShapeDtypeStruct((n, 128), jnp.float32),
            jax.ShapeDtypeStruct((n, 1), jnp.float32),
        ],
    )(xp, degp2, in_w, in_b, bng, bnb, w0)


def _tc_mid(accp, g, hprev, dinv, b_gcn, bng, bnb, w_next, n, r):
    nblk = n // r
    bns = 1.0 / float(_np.sqrt(1.0 + _EPS))
    residual = hprev is not None

    def body(*refs):
        if residual:
            (acc_ref, g_ref, h_ref, dinv_ref, b_ref, bg_ref, bb_ref, w_ref,
             h_out, g_out) = refs
        else:
            (acc_ref, g_ref, dinv_ref, b_ref, bg_ref, bb_ref, w_ref,
             h_out, g_out) = refs
        m = acc_ref[0] + acc_ref[1]
        dinv = dinv_ref[...]
        conv = dinv * (m + g_ref[...]) + b_ref[...]
        hn = jnp.maximum(conv * (bg_ref[...] * bns) + bb_ref[...], 0.0)
        if residual:
            hn = hn + h_ref[...]
        h_out[...] = hn
        g_out[...] = jnp.dot(dinv * hn, w_ref[...],
                             preferred_element_type=jnp.float32)

    in_specs = [pl.BlockSpec((2, r, 128), lambda i: (0, i, 0)),
                pl.BlockSpec((r, 128), lambda i: (i, 0))]
    args = [accp, g]
    if residual:
        in_specs.append(pl.BlockSpec((r, 128), lambda i: (i, 0)))
        args.append(hprev)
    in_specs += [
        pl.BlockSpec((r, 1), lambda i: (i, 0)),
        pl.BlockSpec((1, 128), lambda i: (0, 0)),
        pl.BlockSpec((1, 128), lambda i: (0, 0)),
        pl.BlockSpec((1, 128), lambda i: (0, 0)),
        pl.BlockSpec((128, 128), lambda i: (0, 0)),
    ]
    args += [dinv, b_gcn, bng, bnb, w_next]
    return pl.pallas_call(
        body,
        grid=(nblk,),
        in_specs=in_specs,
        out_specs=[
            pl.BlockSpec((r, 128), lambda i: (i, 0)),
            pl.BlockSpec((r, 128), lambda i: (i, 0)),
        ],
        out_shape=[
            jax.ShapeDtypeStruct((n, 128), jnp.float32),
            jax.ShapeDtypeStruct((n, 128), jnp.float32),
        ],
    )(*args)


def _tc_final(accp, g, hprev, dinv, batch_r, b_gcn, bng, bnb, cls, n, r,
              num_graphs):
    nblk = n // r
    bns = 1.0 / float(_np.sqrt(1.0 + _EPS))
    (c0w, c0b, c0g, c0bb, c1w, c1b, c1g, c1bb, c2w, c2b) = cls

    def body(acc_ref, g_ref, h_ref, dinv_ref, b_ref, bg_ref, bb_ref,
             batch_ref, c0w_r, c0b_r, c0g_r, c0bb_r, c1w_r, c1b_r, c1g_r,
             c1bb_r, c2w_r, c2b_r, out_ref, sums, counts):
        i = pl.program_id(0)

        @pl.when(i == 0)
        def _init():
            sums[...] = jnp.zeros_like(sums)
            counts[...] = jnp.zeros_like(counts)

        m = acc_ref[0] + acc_ref[1]
        dinv = dinv_ref[...]
        conv = dinv * (m + g_ref[...]) + b_ref[...]
        hn = jnp.maximum(conv * (bg_ref[...] * bns) + bb_ref[...], 0.0)
        h3 = hn + h_ref[...]

        seg = batch_ref[0]
        gid = lax.broadcasted_iota(jnp.int32, (num_graphs, r), 0)
        onehot = jnp.where(gid == seg, 1.0, 0.0)
        sums[...] += jnp.dot(onehot, h3, preferred_element_type=jnp.float32)
        cnt = jnp.sum(onehot, axis=1, keepdims=True)
        counts[...] += jnp.broadcast_to(cnt, counts.shape)

        @pl.when(i == nblk - 1)
        def _fin():
            gemb = sums[...] / jnp.maximum(counts[...], 1.0)
            z = jnp.dot(gemb, c0w_r[...], preferred_element_type=jnp.float32)
            z = (z + c0b_r[...]) * (c0g_r[...] * bns) + c0bb_r[...]
            z = jnp.maximum(z, 0.0)
            z = jnp.dot(z, c1w_r[...], preferred_element_type=jnp.float32)
            z = (z + c1b_r[...]) * (c1g_r[...] * bns) + c1bb_r[...]
            z = jnp.maximum(z, 0.0)
            z = jnp.dot(z, c2w_r[...], preferred_element_type=jnp.float32)
            out_ref[...] = z + c2b_r[...]

    full = lambda a: pl.BlockSpec(a.shape, lambda i: tuple(0 for _ in a.shape))
    return pl.pallas_call(
        body,
        grid=(nblk,),
        in_specs=[
            pl.BlockSpec((2, r, 128), lambda i: (0, i, 0)),
            pl.BlockSpec((r, 128), lambda i: (i, 0)),
            pl.BlockSpec((r, 128), lambda i: (i, 0)),
            pl.BlockSpec((r, 1), lambda i: (i, 0)),
            pl.BlockSpec((1, 128), lambda i: (0, 0)),
            pl.BlockSpec((1, 128), lambda i: (0, 0)),
            pl.BlockSpec((1, 128), lambda i: (0, 0)),
            pl.BlockSpec((1, 1, r), lambda i: (i, 0, 0)),
            full(c0w), full(c0b), full(c0g), full(c0bb),
            full(c1w), full(c1b), full(c1g), full(c1bb),
            full(c2w), full(c2b),
        ],
        out_specs=pl.BlockSpec((num_graphs, 2), lambda i: (0, 0)),
        out_shape=jax.ShapeDtypeStruct((num_graphs, 2), jnp.float32),
        scratch_shapes=[
            pltpu.VMEM((num_graphs, 128), jnp.float32),
            pltpu.VMEM((num_graphs, 128), jnp.float32),
        ],
    )(accp, g, hprev, dinv, b_gcn, bng, bnb, batch_r,
      c0w, c0b, c0g, c0bb, c1w, c1b, c1g, c1bb, c2w, c2b)


def kernel(x, edge_index, batch, params):
    n, in_dim = x.shape
    e = edge_index.shape[1]
    num_graphs = 32
    r = 1000
    ept = e // _NW
    k = (ept + 127) // 128
    pad = k * 128 - ept
    rpt = n // _NS
    npad = ((n + 639) // 640) * 640

    src = edge_index[0].astype(jnp.int32)
    dst = edge_index[1].astype(jnp.int32)
    src_r = jnp.pad(src.reshape(_NW, ept), ((0, 0), (0, pad))).reshape(
        _NW, k, 128)
    dst_r = jnp.pad(dst.reshape(_NW, ept), ((0, 0), (0, pad)),
                    constant_values=n).reshape(_NW, k, 128)
    ones128 = jnp.ones((128, 128), jnp.float32)
    z64 = jnp.zeros((64, 128), jnp.float32)
    z128 = jnp.zeros((128, 128), jnp.float32)

    p = params
    row = lambda v: v.reshape(1, -1)

    degp = _sc_degrees(dst_r, ones128, z64, n, npad, rpt)[:, :, :1]
    degp2 = jnp.transpose(degp, (1, 0, 2)).reshape(npad, _NC)

    xp = jnp.pad(x, ((0, 0), (0, (-in_dim) % 8)))
    in_w = jnp.pad(p['in_W'], ((0, (-in_dim) % 8), (0, 0)))
    g0, dinv = _tc_pre(xp, degp2, in_w, row(p['in_b']), row(p['bn_in_g']),
                       row(p['bn_in_b']), p['gcn0_W'], n, r)

    acc = _sc_scatter(g0, src_r, dst_r, z128, n, npad, rpt)
    h1, g1 = _tc_mid(acc, g0, None, dinv, row(p['gcn0_b']), row(p['bn0_g']),
                     row(p['bn0_b']), p['gcn1_W'], n, r)

    acc = _sc_scatter(g1, src_r, dst_r, z128, n, npad, rpt)
    h2, g2 = _tc_mid(acc, g1, h1, dinv, row(p['gcn1_b']), row(p['bn1_g']),
                     row(p['bn1_b']), p['gcn2_W'], n, r)

    acc = _sc_scatter(g2, src_r, dst_r, z128, n, npad, rpt)
    batch_r = batch.astype(jnp.int32).reshape(n // r, 1, r)
    cls = (p['cl0_W'], row(p['cl0_b']), row(p['cbn0_g']), row(p['cbn0_b']),
           p['cl1_W'], row(p['cl1_b']), row(p['cbn1_g']), row(p['cbn1_b']),
           p['cl2_W'], row(p['cl2_b']))
    return _tc_final(acc, g2, h2, dinv, batch_r, row(p['gcn2_b']),
                     row(p['bn2_g']), row(p['bn2_b']), cls, n, r, num_graphs)
